# baseline (speedup 1.0000x reference)
import functools

import jax
import jax.numpy as jnp
from jax.experimental import pallas as pl
from jax.experimental.pallas import tpu as pltpu


def _body(adj_ref, w_ref, idx_ref, out_ref, row_ref, *, row_tile, seq_l, k):
    h = pl.program_id(0)
    r = pl.program_id(1)

    @pl.when(r == 0)
    def _compute_row():
        mean_adj = jnp.mean(adj_ref[...], axis=0)
        m_hj = jax.lax.dot_general(
            w_ref[...], mean_adj, (((0,), (1,)), ((), ())),
            preferred_element_type=jnp.float32)
        cols = jax.lax.broadcasted_iota(jnp.int32, (k, seq_l), 1)
        mask = (cols == idx_ref[...]).astype(jnp.float32)
        row_ref[...] = jax.lax.dot_general(
            m_hj, mask, (((1,), (0,)), ((), ())),
            preferred_element_type=jnp.float32)

    out_ref[0, 0] = jnp.broadcast_to(row_ref[pl.ds(h, 1), :], (row_tile, seq_l))


def _bias_for_batch(adj_b, idx_b, w, seq_l, row_tile, interpret=False):
    k_i, k_j, r_dim = adj_b.shape
    h_dim = w.shape[1]
    grid = (h_dim, seq_l // row_tile)
    out = pl.pallas_call(
        functools.partial(_body, row_tile=row_tile, seq_l=seq_l, k=k_j),
        grid=grid,
        in_specs=[
            pl.BlockSpec((k_i, k_j, r_dim), lambda h, r: (0, 0, 0)),
            pl.BlockSpec(w.shape, lambda h, r: (0, 0)),
            pl.BlockSpec((k_j, 1), lambda h, r: (0, 0)),
        ],
        out_specs=pl.BlockSpec((1, 1, row_tile, seq_l),
                               lambda h, r: (h, 0, r, 0)),
        out_shape=jax.ShapeDtypeStruct((h_dim, 1, seq_l, seq_l), adj_b.dtype),
        scratch_shapes=[pltpu.VMEM((h_dim, seq_l), jnp.float32)],
        compiler_params=pltpu.CompilerParams(
            dimension_semantics=("parallel", "arbitrary")),
        interpret=interpret,
    )(adj_b, w, idx_b)
    return out[:, 0]


def kernel(adj_matrix, top_k_indices, seq_l, relation_head_weights,
           interpret=False):
    b = adj_matrix.shape[0]
    seq_l = 2048
    row_tile = 512
    outs = []
    for bi in range(b):
        idx_col = top_k_indices[bi].astype(jnp.int32).reshape(-1, 1)
        outs.append(_bias_for_batch(adj_matrix[bi], idx_col,
                                    relation_head_weights, seq_l, row_tile,
                                    interpret=interpret))
    return jnp.stack(outs, axis=0)


# speedup vs baseline: 1.0950x; 1.0950x over previous
import functools

import jax
import jax.numpy as jnp
from jax.experimental import pallas as pl
from jax.experimental.pallas import tpu as pltpu


def _body(adj_ref, w_ref, idx_ref, out_ref, row_ref, *, row_tile, seq_l, k):
    h = pl.program_id(0)
    r = pl.program_id(1)

    @pl.when((h == 0) & (r == 0))
    def _compute_row():
        mean_adj = jnp.mean(adj_ref[...], axis=0)
        m_hj = jax.lax.dot_general(
            w_ref[...], mean_adj, (((0,), (1,)), ((), ())),
            preferred_element_type=jnp.float32)
        cols = jax.lax.broadcasted_iota(jnp.int32, (k, seq_l), 1)
        mask = (cols == idx_ref[...]).astype(jnp.float32)
        row_ref[...] = jax.lax.dot_general(
            m_hj, mask, (((1,), (0,)), ((), ())),
            preferred_element_type=jnp.float32)

    out_ref[0, 0] = jnp.broadcast_to(row_ref[pl.ds(h, 1), :], (row_tile, seq_l))


def _bias_for_batch(adj_b, idx_b, w, seq_l, row_tile, interpret=False):
    k_i, k_j, r_dim = adj_b.shape
    h_dim = w.shape[1]
    grid = (h_dim, seq_l // row_tile)
    out = pl.pallas_call(
        functools.partial(_body, row_tile=row_tile, seq_l=seq_l, k=k_j),
        grid=grid,
        in_specs=[
            pl.BlockSpec((k_i, k_j, r_dim), lambda h, r: (0, 0, 0)),
            pl.BlockSpec(w.shape, lambda h, r: (0, 0)),
            pl.BlockSpec((k_j, 1), lambda h, r: (0, 0)),
        ],
        out_specs=pl.BlockSpec((1, 1, row_tile, seq_l),
                               lambda h, r: (h, 0, r, 0)),
        out_shape=jax.ShapeDtypeStruct((h_dim, 1, seq_l, seq_l), adj_b.dtype),
        scratch_shapes=[pltpu.VMEM((h_dim, seq_l), jnp.float32)],
        interpret=interpret,
    )(adj_b, w, idx_b)
    return out[:, 0]


def kernel(adj_matrix, top_k_indices, seq_l, relation_head_weights,
           interpret=False):
    b = adj_matrix.shape[0]
    seq_l = 2048
    row_tile = 512
    outs = []
    for bi in range(b):
        idx_col = top_k_indices[bi].astype(jnp.int32).reshape(-1, 1)
        outs.append(_bias_for_batch(adj_matrix[bi], idx_col,
                                    relation_head_weights, seq_l, row_tile,
                                    interpret=interpret))
    return jnp.stack(outs, axis=0)


# speedup vs baseline: 1.0987x; 1.0034x over previous
import functools

import jax
import jax.numpy as jnp
from jax.experimental import pallas as pl
from jax.experimental.pallas import tpu as pltpu


def _body(adj_ref, w_ref, idx_ref, out_ref, row_ref, buf_ref, sem,
          *, row_tile, seq_l, k, h_dim):
    mean_adj = jnp.mean(adj_ref[...], axis=0)
    m_hj = jax.lax.dot_general(
        w_ref[...], mean_adj, (((0,), (1,)), ((), ())),
        preferred_element_type=jnp.float32)
    cols = jax.lax.broadcasted_iota(jnp.int32, (k, seq_l), 1)
    mask = (cols == idx_ref[...]).astype(jnp.float32)
    row_ref[...] = jax.lax.dot_general(
        m_hj, mask, (((1,), (0,)), ((), ())),
        preferred_element_type=jnp.float32)

    n_tiles = seq_l // row_tile

    for h in range(h_dim):
        slot = h % 2
        if h >= 2:
            for t in range(n_tiles):
                pltpu.make_async_copy(
                    buf_ref.at[slot],
                    out_ref.at[h - 2, pl.ds(t * row_tile, row_tile)],
                    sem.at[slot]).wait()
        buf_ref[slot] = jnp.broadcast_to(row_ref[pl.ds(h, 1), :],
                                         (row_tile, seq_l))
        for t in range(n_tiles):
            pltpu.make_async_copy(
                buf_ref.at[slot],
                out_ref.at[h, pl.ds(t * row_tile, row_tile)],
                sem.at[slot]).start()

    for h in (h_dim - 2, h_dim - 1):
        slot = h % 2
        for t in range(n_tiles):
            pltpu.make_async_copy(
                buf_ref.at[slot],
                out_ref.at[h, pl.ds(t * row_tile, row_tile)],
                sem.at[slot]).wait()


def _bias_for_batch(adj_b, idx_b, w, seq_l, row_tile, interpret=False):
    k_i, k_j, r_dim = adj_b.shape
    h_dim = w.shape[1]
    return pl.pallas_call(
        functools.partial(_body, row_tile=row_tile, seq_l=seq_l, k=k_j,
                          h_dim=h_dim),
        in_specs=[
            pl.BlockSpec(memory_space=pltpu.VMEM),
            pl.BlockSpec(memory_space=pltpu.VMEM),
            pl.BlockSpec(memory_space=pltpu.VMEM),
        ],
        out_specs=pl.BlockSpec(memory_space=pl.ANY),
        out_shape=jax.ShapeDtypeStruct((h_dim, seq_l, seq_l), adj_b.dtype),
        scratch_shapes=[
            pltpu.VMEM((h_dim, seq_l), jnp.float32),
            pltpu.VMEM((2, row_tile, seq_l), jnp.float32),
            pltpu.SemaphoreType.DMA((2,)),
        ],
        interpret=interpret,
    )(adj_b, w, idx_b)


def kernel(adj_matrix, top_k_indices, seq_l, relation_head_weights,
           interpret=False):
    b = adj_matrix.shape[0]
    seq_l = 2048
    row_tile = 512
    outs = []
    for bi in range(b):
        idx_col = top_k_indices[bi].astype(jnp.int32).reshape(-1, 1)
        outs.append(_bias_for_batch(adj_matrix[bi], idx_col,
                                    relation_head_weights, seq_l, row_tile,
                                    interpret=interpret))
    return jnp.stack(outs, axis=0)
